# baseline (speedup 1.0000x reference)
import functools

import jax
import jax.numpy as jnp
from jax import lax
from jax.experimental import pallas as pl
from jax.experimental.pallas import tpu as pltpu
from jax.experimental.pallas import tpu_sc as plsc

_NC = 2
_NS = 16
_NW = _NC * _NS
_L = 16

_B = 16384
_R = 32
_BPW = _B // _NW
_IPW = 3 * _BPW
_GW = 2 * _BPW * _R
_EW = _BPW * _R


_CH = 256
_CW = _CH * _R


def _sc_body(idx_hbm, eps_hbm, mu0, mu1, mu2, lam0, lam1, lam2,
             u_hbm, idx_v, mu_g, lam_g, eps_v, u_v, sem_g, sem_e):
    wid = lax.axis_index("s") * _NC + lax.axis_index("c")
    mus = [mu0, mu1, mu2]
    lams = [lam0, lam1, lam2]
    pltpu.sync_copy(idx_hbm.at[pl.ds(wid * _IPW, _IPW)], idx_v)
    for m in range(3):
        for h in range(_BPW // _CH):
            woff = (wid * 3 + m) * _EW + h * _CW
            cp_e = pltpu.async_copy(
                eps_hbm.at[pl.ds(woff, _CW)], eps_v, sem_e)

            def fire(g, _, mu_tab, lam_tab, base):
                vec = idx_v[pl.ds(base + g * _L, _L)]
                for u in range(_L):
                    idx = vec[u]
                    j = g * _L + u
                    pltpu.async_copy(mu_tab.at[idx], mu_g.at[j], sem_g)
                    pltpu.async_copy(lam_tab.at[idx], lam_g.at[j], sem_g)
                return 0

            lax.fori_loop(
                0, _CH // _L,
                functools.partial(fire, mu_tab=mus[m], lam_tab=lams[m],
                                  base=m * _BPW + h * _CH),
                0)
            pltpu.make_async_copy(
                mus[m].at[pl.ds(0, _CH)], mu_g, sem_g).wait()
            pltpu.make_async_copy(
                lams[m].at[pl.ds(0, _CH)], lam_g, sem_g).wait()
            cp_e.wait()

            def compute(r, _):
                for c in range(_R // _L):
                    sl = pl.ds(c * _L, _L)
                    fl = pl.ds(r * _R + c * _L, _L)
                    u_v[fl] = (mu_g[r, sl]
                               + eps_v[fl] * jnp.exp(lam_g[r, sl] * 0.5))
                return 0

            lax.fori_loop(0, _CH, compute, 0, unroll=4)
            pltpu.sync_copy(
                u_v, u_hbm.at[pl.ds((m * _B + wid * _BPW) * _R + h * _CW,
                                    _CW)])


_sc_gather_reparam = functools.partial(
    pl.kernel,
    out_type=jax.ShapeDtypeStruct((3 * _B * _R,), jnp.float32),
    mesh=plsc.VectorSubcoreMesh(core_axis_name="c", subcore_axis_name="s"),
    scratch_types=[
        pltpu.VMEM((_IPW,), jnp.int32),
        pltpu.VMEM((_CH, _R), jnp.float32),
        pltpu.VMEM((_CH, _R), jnp.float32),
        pltpu.VMEM((_CW,), jnp.float32),
        pltpu.VMEM((_CW,), jnp.float32),
        pltpu.SemaphoreType.DMA,
        pltpu.SemaphoreType.DMA,
    ],
    compiler_params=pltpu.CompilerParams(needs_layout_passes=True),
)(_sc_body)


_TB = 2048


def _mlp_body(u0, u1, u2, w_t, b_in, w_out, b_out, out_ref):
    u = jnp.concatenate([u0[...], u1[...], u2[...]], axis=1)
    h = jnp.dot(u, w_t[...], preferred_element_type=jnp.float32) + b_in[...]
    h = jnp.tanh(h)
    out_ref[...] = (
        jnp.dot(h, w_out[...], preferred_element_type=jnp.float32) + b_out[...]
    )


def _mlp(u0, u1, u2, w_t, b_in, w_out, b_out):
    return pl.pallas_call(
        _mlp_body,
        grid=(_B // _TB,),
        in_specs=[
            pl.BlockSpec((_TB, _R), lambda i: (i, 0)),
            pl.BlockSpec((_TB, _R), lambda i: (i, 0)),
            pl.BlockSpec((_TB, _R), lambda i: (i, 0)),
            pl.BlockSpec((3 * _R, 128), lambda i: (0, 0)),
            pl.BlockSpec((1, 128), lambda i: (0, 0)),
            pl.BlockSpec((128, 2), lambda i: (0, 0)),
            pl.BlockSpec((1, 2), lambda i: (0, 0)),
        ],
        out_specs=pl.BlockSpec((_TB, 2), lambda i: (i, 0)),
        out_shape=jax.ShapeDtypeStruct((_B, 2), jnp.float32),
    )(u0, u1, u2, w_t, b_in, w_out, b_out)


def kernel(x, mu0, mu1, mu2, lam0, lam1, lam2, eps0, eps1, eps2,
           W_in, b_in, W_mean, b_mean, W_lv, b_lv):
    xi = x.astype(jnp.int32)
    idx_all = xi.reshape(_NW, _BPW, 3).transpose(0, 2, 1).reshape(-1)
    eps_all = (jnp.stack([eps0, eps1, eps2], axis=0)
               .reshape(3, _NW, _BPW, _R)
               .transpose(1, 0, 2, 3)
               .reshape(-1))
    u_flat = _sc_gather_reparam(idx_all, eps_all,
                                mu0, mu1, mu2, lam0, lam1, lam2)
    us = u_flat.reshape(3, _B, _R)
    w_t = W_in.T
    w_out = jnp.concatenate([W_mean, W_lv], axis=0).T
    b_out = jnp.concatenate([b_mean, b_lv]).reshape(1, 2)
    out = _mlp(us[0], us[1], us[2], w_t, b_in.reshape(1, 128), w_out, b_out)
    return (out[:, 0:1], out[:, 1:2])


# speedup vs baseline: 1.1263x; 1.1263x over previous
import functools

import jax
import jax.numpy as jnp
from jax import lax
from jax.experimental import pallas as pl
from jax.experimental.pallas import tpu as pltpu
from jax.experimental.pallas import tpu_sc as plsc

_NC = 2
_NS = 16
_NW = _NC * _NS
_L = 16

_B = 16384
_R = 32
_BPW = _B // _NW
_CH = 256


def _sc_body(idx_hbm, eps0, eps1, eps2, mu0, mu1, mu2, lam0, lam1, lam2,
             u0, u1, u2, idx_v, mu_g, lam_g, eps_v, sem_g, sem_e):
    wid = lax.axis_index("s") * _NC + lax.axis_index("c")
    mus = [mu0, mu1, mu2]
    lams = [lam0, lam1, lam2]
    epss = [eps0, eps1, eps2]
    us = [u0, u1, u2]
    for m in range(3):
        for h in range(_BPW // _CH):
            row0 = wid * _BPW + h * _CH
            pltpu.sync_copy(idx_hbm.at[pl.ds(m * _B + row0, _CH)], idx_v)
            cp_e = pltpu.async_copy(
                epss[m].at[pl.ds(row0, _CH)], eps_v, sem_e)

            def fire(g, _, mu_tab, lam_tab):
                vec = idx_v[pl.ds(g * _L, _L)]
                for u in range(_L):
                    idx = vec[u]
                    j = g * _L + u
                    pltpu.async_copy(mu_tab.at[idx], mu_g.at[j], sem_g)
                    pltpu.async_copy(lam_tab.at[idx], lam_g.at[j], sem_g)
                return 0

            lax.fori_loop(
                0, _CH // _L,
                functools.partial(fire, mu_tab=mus[m], lam_tab=lams[m]),
                0)
            pltpu.make_async_copy(
                mus[m].at[pl.ds(0, _CH)], mu_g, sem_g).wait()
            pltpu.make_async_copy(
                lams[m].at[pl.ds(0, _CH)], lam_g, sem_g).wait()
            cp_e.wait()

            def compute(r, _):
                for c in range(_R // _L):
                    sl = pl.ds(c * _L, _L)
                    eps_v[r, sl] = (mu_g[r, sl]
                                    + eps_v[r, sl]
                                    * jnp.exp(lam_g[r, sl] * 0.5))
                return 0

            lax.fori_loop(0, _CH, compute, 0, unroll=4)
            pltpu.sync_copy(eps_v, us[m].at[pl.ds(row0, _CH)])


_sc_gather_reparam = functools.partial(
    pl.kernel,
    out_type=[jax.ShapeDtypeStruct((_B, _R), jnp.float32) for _ in range(3)],
    mesh=plsc.VectorSubcoreMesh(core_axis_name="c", subcore_axis_name="s"),
    scratch_types=[
        pltpu.VMEM((_CH,), jnp.int32),
        pltpu.VMEM((_CH, _R), jnp.float32),
        pltpu.VMEM((_CH, _R), jnp.float32),
        pltpu.VMEM((_CH, _R), jnp.float32),
        pltpu.SemaphoreType.DMA,
        pltpu.SemaphoreType.DMA,
    ],
)(_sc_body)


_TB = 2048


def _mlp_body(u0, u1, u2, w_t, b_in, w_m, w_l, b_m, b_l, mean_ref, lv_ref):
    u = jnp.concatenate([u0[...], u1[...], u2[...]], axis=1)
    h = jnp.dot(u, w_t[...], preferred_element_type=jnp.float32) + b_in[...]
    h = jnp.tanh(h)
    mean_ref[...] = (
        jnp.dot(h, w_m[...], preferred_element_type=jnp.float32) + b_m[...])
    lv_ref[...] = (
        jnp.dot(h, w_l[...], preferred_element_type=jnp.float32) + b_l[...])


def _mlp(u0, u1, u2, w_t, b_in, w_m, w_l, b_m, b_l):
    return pl.pallas_call(
        _mlp_body,
        grid=(_B // _TB,),
        in_specs=[
            pl.BlockSpec((_TB, _R), lambda i: (i, 0)),
            pl.BlockSpec((_TB, _R), lambda i: (i, 0)),
            pl.BlockSpec((_TB, _R), lambda i: (i, 0)),
            pl.BlockSpec((3 * _R, 128), lambda i: (0, 0)),
            pl.BlockSpec((1, 128), lambda i: (0, 0)),
            pl.BlockSpec((128, 1), lambda i: (0, 0)),
            pl.BlockSpec((128, 1), lambda i: (0, 0)),
            pl.BlockSpec((1, 1), lambda i: (0, 0)),
            pl.BlockSpec((1, 1), lambda i: (0, 0)),
        ],
        out_specs=[
            pl.BlockSpec((_TB, 1), lambda i: (i, 0)),
            pl.BlockSpec((_TB, 1), lambda i: (i, 0)),
        ],
        out_shape=[
            jax.ShapeDtypeStruct((_B, 1), jnp.float32),
            jax.ShapeDtypeStruct((_B, 1), jnp.float32),
        ],
    )(u0, u1, u2, w_t, b_in, w_m, w_l, b_m, b_l)


def kernel(x, mu0, mu1, mu2, lam0, lam1, lam2, eps0, eps1, eps2,
           W_in, b_in, W_mean, b_mean, W_lv, b_lv):
    xi = x.astype(jnp.int32)
    idx_all = xi.T.reshape(-1)
    u0, u1, u2 = _sc_gather_reparam(idx_all, eps0, eps1, eps2,
                                    mu0, mu1, mu2, lam0, lam1, lam2)
    w_t = W_in.T
    mean, log_var = _mlp(u0, u1, u2, w_t, b_in.reshape(1, 128),
                         W_mean.T, W_lv.T,
                         b_mean.reshape(1, 1), b_lv.reshape(1, 1))
    return (mean, log_var)
